# baseline (speedup 1.0000x reference)
import functools
import jax
import jax.numpy as jnp
from jax import lax
from jax.experimental import pallas as pl
from jax.experimental.pallas import tpu as pltpu, tpu_sc as plsc

_NEW = 9
_L16 = 16


def _body(B, nz, nq, sf_hbm, zpack_hbm, qpack_hbm, hq_hbm, out_hbm,
          sv, zpk, qpk, hqv, buf):
    info = plsc.get_sparse_core_info()
    NC, NS = info.num_cores, info.num_subcores
    NW = NC * NS
    nb = B // NW
    CH = 128
    nchunks = nb // CH
    wid = lax.axis_index("s") * NC + lax.axis_index("c")
    base = wid * nb

    for j in range(_L16):
        pltpu.sync_copy(sf_hbm.at[pl.ds(j * B + base, nb)], sv.at[j])
    pltpu.sync_copy(zpack_hbm, zpk)
    pltpu.sync_copy(qpack_hbm, qpk)
    pltpu.sync_copy(hq_hbm, hqv)

    zr_v = zpk[0, pl.ds(0, _L16)]
    zc_v = zpk[1, pl.ds(0, _L16)]
    zs_v = zpk[2, pl.ds(0, _L16)]
    qr_a = qpk[0, pl.ds(0, _L16)]
    qr_b = qpk[0, pl.ds(_L16, _L16)]
    qc_a = qpk[1, pl.ds(0, _L16)]
    qc_b = qpk[1, pl.ds(_L16, _L16)]

    zero16 = jnp.zeros((_L16,), dtype=jnp.float32)

    def _zero_r(r, carry):
        for c in range(_NEW):
            for ch in range(6):
                for g in range(CH // _L16):
                    buf[r, c, ch, pl.ds(g * _L16, _L16)] = zero16
        return carry

    lax.fori_loop(0, _NEW, _zero_r, 0)

    def _chunk(ci, carry):
        def _group(g, carry2):
            off = ci * CH + g * _L16

            for k in range(2 * nz):
                ch = 2 if k < nz else 3
                val = 2.0 * sv[zs_v[k], pl.ds(off, _L16)] - 1.0
                buf[zr_v[k], zc_v[k], ch, pl.ds(g * _L16, _L16)] = val
            for k in range(nq):
                qr_k = qr_a[k] if k < _L16 else qr_b[k - _L16]
                qc_k = qc_a[k] if k < _L16 else qc_b[k - _L16]
                row = hqv[k, pl.ds(0, _L16)]
                acc4 = jnp.zeros((_L16,), dtype=jnp.float32)
                acc5 = jnp.zeros((_L16,), dtype=jnp.float32)
                for j in range(nz):
                    acc4 = acc4 + row[j] * sv[j, pl.ds(off, _L16)]
                    acc5 = acc5 + row[nz + j] * sv[nz + j, pl.ds(off, _L16)]
                buf[qr_k, qc_k, 4, pl.ds(g * _L16, _L16)] = acc4
                buf[qr_k, qc_k, 5, pl.ds(g * _L16, _L16)] = acc5
            return carry2

        lax.fori_loop(0, CH // _L16, _group, 0)
        pltpu.sync_copy(buf, out_hbm.at[:, :, :, pl.ds(base + ci * CH, CH)])
        return carry

    lax.fori_loop(0, nchunks, _chunk, 0)


def kernel(syndrome, H_z, H_x, qubit_rows, qubit_cols, qubit_src_idx,
           z_stab_rows, z_stab_cols, z_stab_src_idx,
           x_stab_rows, x_stab_cols, x_stab_src_idx):
    B = syndrome.shape[0]
    nz = H_z.shape[0]
    nq = qubit_rows.shape[0]
    f32 = jnp.float32
    sflat = jnp.reshape(syndrome.T, (_L16 * B,))

    zpack = jnp.stack([
        jnp.concatenate([z_stab_rows, x_stab_rows]),
        jnp.concatenate([z_stab_cols, x_stab_cols]),
        jnp.concatenate([z_stab_src_idx, nz + x_stab_src_idx]),
    ]).astype(jnp.int32)
    qpack = jnp.stack([
        jnp.pad(qubit_rows, (0, 2 * _L16 - nq)),
        jnp.pad(qubit_cols, (0, 2 * _L16 - nq)),
    ]).astype(jnp.int32)
    onehot = (qubit_src_idx[None, :] == jax.lax.iota(jnp.int32, H_z.shape[1])[:, None]).astype(f32)
    hq = jnp.concatenate([(H_z @ onehot).T, (H_x @ onehot).T], axis=1) * 0.25

    mesh = plsc.VectorSubcoreMesh(core_axis_name="c", subcore_axis_name="s")
    NW = 32
    nb = B // NW
    run = pl.kernel(
        functools.partial(_body, B, nz, nq),
        out_type=jax.ShapeDtypeStruct((_NEW, _NEW, 6, B), f32),
        mesh=mesh,
        scratch_types=[
            pltpu.VMEM((_L16, nb), f32),
            pltpu.VMEM((3, _L16), jnp.int32),
            pltpu.VMEM((2, 2 * _L16), jnp.int32),
            pltpu.VMEM((nq, _L16), f32),
            pltpu.VMEM((_NEW, _NEW, 6, 128), f32),
        ],
        compiler_params=pltpu.CompilerParams(use_tc_tiling_on_sc=True),
    )
    outT = run(sflat, zpack, qpack, hq)
    return jnp.transpose(outT, (3, 2, 0, 1))


# speedup vs baseline: 1.2656x; 1.2656x over previous
import functools
import jax
import jax.numpy as jnp
from jax import lax
from jax.experimental import pallas as pl
from jax.experimental.pallas import tpu as pltpu, tpu_sc as plsc

_NEW = 9
_L16 = 16
_CH = 128


def _body(B, nz, nq, s2d_hbm, zpack_hbm, qpack_hbm, hq_hbm, out_hbm,
          sv, zpk, qpk, hqv, buf, sem_s, sem_o):
    info = plsc.get_sparse_core_info()
    NC, NS = info.num_cores, info.num_subcores
    NW = NC * NS
    nb = B // NW
    nchunks = nb // _CH
    wid = lax.axis_index("s") * NC + lax.axis_index("c")
    base = wid * nb

    stage = pltpu.make_async_copy(
        s2d_hbm.at[:, pl.ds(base, nb)], sv, sem_s)
    stage.start()
    pltpu.sync_copy(zpack_hbm, zpk)
    pltpu.sync_copy(qpack_hbm, qpk)
    pltpu.sync_copy(hq_hbm, hqv)

    zero16 = jnp.zeros((_L16,), dtype=jnp.float32)

    def _zero_r(r, carry):
        for c in range(_NEW):
            for ch in range(6):
                for g in range(_CH // _L16):
                    buf[r, c, ch, pl.ds(g * _L16, _L16)] = zero16
        return carry

    lax.fori_loop(0, _NEW, _zero_r, 0)
    stage.wait()

    zr_v = zpk[0, pl.ds(0, _L16)]
    zc_v = zpk[1, pl.ds(0, _L16)]
    zs_v = zpk[2, pl.ds(0, _L16)]
    qr_a = qpk[0, pl.ds(0, _L16)]
    qr_b = qpk[0, pl.ds(_L16, _L16)]
    qc_a = qpk[1, pl.ds(0, _L16)]
    qc_b = qpk[1, pl.ds(_L16, _L16)]

    def _chunk_sync(ci, carry):
        _chunk_compute(ci)
        pltpu.sync_copy(buf, out_hbm.at[:, :, :, pl.ds(base + ci * _CH, _CH)])
        return carry

    def _chunk_compute(ci):
        def _group(g, carry2):
            off = ci * _CH + g * _L16
            s16 = [sv[j, pl.ds(off, _L16)] for j in range(_L16)]

            for k in range(2 * nz):
                ch = 2 if k < nz else 3
                val = 2.0 * sv[zs_v[k], pl.ds(off, _L16)] - 1.0
                buf[zr_v[k], zc_v[k], ch, pl.ds(g * _L16, _L16)] = val
            for k in range(nq):
                qr_k = qr_a[k] if k < _L16 else qr_b[k - _L16]
                qc_k = qc_a[k] if k < _L16 else qc_b[k - _L16]
                row = hqv[k, pl.ds(0, _L16)]
                acc4 = jnp.zeros((_L16,), dtype=jnp.float32)
                acc5 = jnp.zeros((_L16,), dtype=jnp.float32)
                for j in range(nz):
                    acc4 = acc4 + row[j] * s16[j]
                    acc5 = acc5 + row[nz + j] * s16[nz + j]
                buf[qr_k, qc_k, 4, pl.ds(g * _L16, _L16)] = acc4
                buf[qr_k, qc_k, 5, pl.ds(g * _L16, _L16)] = acc5
            return carry2

        lax.fori_loop(0, _CH // _L16, _group, 0)

    lax.fori_loop(0, nchunks, _chunk_sync, 0)


def kernel(syndrome, H_z, H_x, qubit_rows, qubit_cols, qubit_src_idx,
           z_stab_rows, z_stab_cols, z_stab_src_idx,
           x_stab_rows, x_stab_cols, x_stab_src_idx):
    B = syndrome.shape[0]
    nz = H_z.shape[0]
    nq = qubit_rows.shape[0]
    f32 = jnp.float32

    zpack = jnp.stack([
        jnp.concatenate([z_stab_rows, x_stab_rows]),
        jnp.concatenate([z_stab_cols, x_stab_cols]),
        jnp.concatenate([z_stab_src_idx, nz + x_stab_src_idx]),
    ]).astype(jnp.int32)
    qpack = jnp.stack([
        jnp.pad(qubit_rows, (0, 2 * _L16 - nq)),
        jnp.pad(qubit_cols, (0, 2 * _L16 - nq)),
    ]).astype(jnp.int32)
    onehot = (qubit_src_idx[None, :] == jax.lax.iota(jnp.int32, H_z.shape[1])[:, None]).astype(f32)
    hq = jnp.concatenate([(H_z @ onehot).T, (H_x @ onehot).T], axis=1) * 0.25

    mesh = plsc.VectorSubcoreMesh(core_axis_name="c", subcore_axis_name="s")
    NW = 32
    nb = B // NW
    run = pl.kernel(
        functools.partial(_body, B, nz, nq),
        out_type=jax.ShapeDtypeStruct((_NEW, _NEW, 6, B), f32),
        mesh=mesh,
        scratch_types=[
            pltpu.VMEM((_L16, nb), f32),
            pltpu.VMEM((3, _L16), jnp.int32),
            pltpu.VMEM((2, 2 * _L16), jnp.int32),
            pltpu.VMEM((nq, _L16), f32),
            pltpu.VMEM((_NEW, _NEW, 6, _CH), f32),
            pltpu.SemaphoreType.DMA,
            pltpu.SemaphoreType.DMA,
        ],
        compiler_params=pltpu.CompilerParams(use_tc_tiling_on_sc=True),
    )
    outT = run(syndrome.T, zpack, qpack, hq)
    return jnp.transpose(outT, (3, 2, 0, 1))


# speedup vs baseline: 4.3006x; 3.3982x over previous
import jax
import jax.numpy as jnp
from jax.experimental import pallas as pl
from jax.experimental.pallas import tpu as pltpu

_NEW = 9


def _body(s_ref, hz_ref, hx_ref, qr, qc, qs, zr, zc, zs, xr, xc, xs,
          o_ref, mt):
    i = pl.program_id(0)

    @pl.when(i == 0)
    def _build():
        mt[...] = jnp.zeros(mt.shape, dtype=mt.dtype)
        lane = jax.lax.iota(jnp.int32, 17)
        bias_row = jnp.where(lane == 16, -1.0, 0.0).astype(mt.dtype)
        for k in range(zr.shape[0]):
            row = jnp.where(lane == zs[k], 2.0, 0.0).astype(mt.dtype)
            mt[zr[k], zc[k], 2] = row + bias_row
        for k in range(xr.shape[0]):
            row = jnp.where(lane == 8 + xs[k], 2.0, 0.0).astype(mt.dtype)
            mt[xr[k], xc[k], 3] = row + bias_row
        for k in range(qr.shape[0]):
            row4 = jnp.zeros((17,), dtype=mt.dtype)
            row5 = jnp.zeros((17,), dtype=mt.dtype)
            for j in range(hz_ref.shape[0]):
                row4 = jnp.where(lane == j, hz_ref[j, qs[k]] * 0.25, row4)
                row5 = jnp.where(lane == 8 + j, hx_ref[j, qs[k]] * 0.25, row5)
            mt[qr[k], qc[k], 4] = row4
            mt[qr[k], qc[k], 5] = row5

    s = s_ref[...]
    for c in range(_NEW):
        o_ref[0, c] = jax.lax.dot_general(
            mt[i, c, :, : s.shape[0]], s, (((1,), (0,)), ((), ())),
            preferred_element_type=jnp.float32) + mt[i, c, :, s.shape[0]:]


def kernel(syndrome, H_z, H_x, qubit_rows, qubit_cols, qubit_src_idx,
           z_stab_rows, z_stab_cols, z_stab_src_idx,
           x_stab_rows, x_stab_cols, x_stab_src_idx):
    B = syndrome.shape[0]
    nsyn = H_z.shape[0] + H_x.shape[0]
    sT = syndrome.T
    smem = pl.BlockSpec(memory_space=pltpu.MemorySpace.SMEM)
    outT = pl.pallas_call(
        _body,
        grid=(_NEW,),
        in_specs=[pl.BlockSpec((nsyn, B), lambda i: (0, 0))] + [smem] * 11,
        out_specs=pl.BlockSpec((1, _NEW, 6, B), lambda i: (i, 0, 0, 0)),
        out_shape=jax.ShapeDtypeStruct((_NEW, _NEW, 6, B), syndrome.dtype),
        scratch_shapes=[pltpu.VMEM((_NEW, _NEW, 6, nsyn + 1), syndrome.dtype)],
    )(sT, H_z, H_x, qubit_rows, qubit_cols, qubit_src_idx,
      z_stab_rows, z_stab_cols, z_stab_src_idx,
      x_stab_rows, x_stab_cols, x_stab_src_idx)
    return jnp.transpose(outT, (3, 2, 0, 1))
